# baseline (speedup 1.0000x reference)
import functools

import jax
import jax.numpy as jnp
from jax import lax
from jax.experimental import pallas as pl
from jax.experimental.pallas import tpu as pltpu
from jax.experimental.pallas import tpu_sc as plsc

_NC = 2
_NS = 16
_L = 16
_NW = _NC * _NS

_CHUNK = 8
_DEG = 32
_SLOTS = _CHUNK * _DEG
_G = _SLOTS // 128


def _sc_aggregate(nodes2, ne2, src_tab, dst_tab, w_tab, num_units, n_per_dir):
    D = nodes2.shape[1]
    nvec = D // _L
    num_chunks = num_units // _CHUNK
    chunks_per_dir = n_per_dir // _CHUNK
    max_k = (num_chunks + _NW - 1) // _NW + 2
    n_t = (max_k + 1) // 2 + 1

    mesh = plsc.VectorSubcoreMesh(
        core_axis_name="c", subcore_axis_name="s",
        num_cores=_NC, num_subcores=_NS)

    @functools.partial(
        pl.kernel,
        out_type=jax.ShapeDtypeStruct((num_units, D), jnp.float32),
        mesh=mesh,
        scratch_types=[
            pltpu.VMEM((2, _G, 128), jnp.int32),
            pltpu.VMEM((2, _G, 128), jnp.int32),
            pltpu.VMEM((2, _SLOTS), jnp.float32),
            pltpu.VMEM((2, _SLOTS), jnp.float32),
            pltpu.VMEM((2, _SLOTS, 128), jnp.float32),
            pltpu.VMEM((2, _CHUNK, 128), jnp.float32),
        ] + [pltpu.SemaphoreType.DMA] * 8,
    )
    def k(nodes_hbm, ne_hbm, src_hbm, dst_hbm, w_hbm, out_hbm,
          eidx_v, ids_v, wland_v, wacc_v, rows_v, out_v,
          sA0, sA1, sB0, sB1, sC0, sC1, sE0, sE1):
        wid = lax.axis_index("s") * _NC + lax.axis_index("c")
        sA = (sA0, sA1)
        sB = (sB0, sB1)
        sC = (sC0, sC1)
        sE = (sE0, sE1)

        def cid_of(kk):
            return wid + kk * _NW

        def local_valid(kk):
            return cid_of(kk) < num_chunks

        def stage_a(kk, p):
            cid = cid_of(kk)

            @pl.when(local_valid(kk))
            def _():
                pltpu.async_copy(
                    ne_hbm.at[pl.ds(cid * _G, _G), :], eidx_v.at[p], sA[p])

        def wait_a(kk, p):
            @pl.when(local_valid(kk))
            def _():
                pltpu.make_async_copy(
                    ne_hbm.at[pl.ds(0, _G), :], eidx_v.at[p], sA[p]).wait()

        def stage_b(kk, p):
            cid = cid_of(kk)
            valid = local_valid(kk)
            dcol = (cid // chunks_per_dir) % 2

            def issue(tab_hbm):
                for g in range(_G):
                    pltpu.async_copy(
                        tab_hbm.at[eidx_v.at[p, g]], ids_v.at[p, g], sB[p])
                for g in range(_G):
                    pltpu.async_copy(
                        w_hbm.at[eidx_v.at[p, g]],
                        wland_v.at[p, pl.ds(g * 128, 128)], sB[p])

            @pl.when(valid & (dcol == 0))
            def _():
                issue(src_hbm)

            @pl.when(valid & (dcol == 1))
            def _():
                issue(dst_hbm)

        def wait_b(kk, p):
            @pl.when(local_valid(kk))
            def _():
                for g in range(_G):
                    pltpu.make_async_copy(
                        src_hbm.at[pl.ds(0, 128)], ids_v.at[p, g],
                        sB[p]).wait()
                for g in range(_G):
                    pltpu.make_async_copy(
                        w_hbm.at[pl.ds(0, 128)],
                        wland_v.at[p, pl.ds(g * 128, 128)], sB[p]).wait()

        def wcopy(kk, p):
            @pl.when(local_valid(kk))
            def _():
                for j in range(_SLOTS // _L):
                    wacc_v[p, pl.ds(j * _L, _L)] = (
                        wland_v[p, pl.ds(j * _L, _L)])

        def stage_c(kk, p):
            @pl.when(local_valid(kk))
            def _():
                for g in range(_G):
                    pltpu.async_copy(
                        nodes_hbm.at[ids_v.at[p, g]],
                        rows_v.at[p, pl.ds(g * 128, 128), :], sC[p])

        def wait_c(kk, p):
            @pl.when(local_valid(kk))
            def _():
                for g in range(_G):
                    pltpu.make_async_copy(
                        nodes_hbm.at[pl.ds(0, 128), :],
                        rows_v.at[p, pl.ds(g * 128, 128), :], sC[p]).wait()

        def wait_e(kk, p, extra_pred):
            @pl.when(extra_pred & local_valid(kk))
            def _():
                pltpu.make_async_copy(
                    out_hbm.at[pl.ds(0, _CHUNK), :], out_v.at[p],
                    sE[p]).wait()

        def acc_e(kk, p):
            cid = cid_of(kk)

            @pl.when(local_valid(kk))
            def _():
                def unit_body(i, _):
                    wa = wacc_v[p, pl.ds(i * _DEG, _L)]
                    wb = wacc_v[p, pl.ds(i * _DEG + _L, _L)]

                    def edge_body(d, acc):
                        s0 = i * _DEG + d
                        iv = jnp.broadcast_to(d, (_L,))
                        wva = jnp.take_along_axis(
                            wa, iv, axis=0, mode="promise_in_bounds")
                        wvb = jnp.take_along_axis(
                            wb, iv, axis=0, mode="promise_in_bounds")
                        return tuple(
                            acc[v]
                            + wva * rows_v[p, s0, pl.ds(v * _L, _L)]
                            + wvb * rows_v[p, s0 + _L, pl.ds(v * _L, _L)]
                            for v in range(nvec))
                    zero = jnp.zeros((_L,), jnp.float32)
                    acc = lax.fori_loop(
                        0, _L, edge_body, tuple(zero for _ in range(nvec)),
                        unroll=4)
                    for v in range(nvec):
                        out_v[p, i, pl.ds(v * _L, _L)] = acc[v]
                    return 0
                lax.fori_loop(0, _CHUNK, unit_body, 0)
                pltpu.async_copy(
                    out_v.at[p],
                    out_hbm.at[pl.ds(cid * _CHUNK, _CHUNK), :], sE[p])

        stage_a(0, 0)
        stage_a(1, 1)
        wait_a(0, 0)
        stage_b(0, 0)
        wait_b(0, 0)
        wcopy(0, 0)
        stage_c(0, 0)
        wait_a(1, 1)
        stage_b(1, 1)

        def loop_body(t, _):
            for m in range(2):
                kk = 2 * t + m
                p0 = m
                p1 = 1 - m
                stage_a(kk + 2, p0)
                wait_b(kk + 1, p1)
                wcopy(kk + 1, p1)
                stage_c(kk + 1, p1)
                wait_c(kk, p0)
                wait_a(kk + 2, p0)
                stage_b(kk + 2, p0)
                wait_e(kk - 2, p0, t >= 1)
                acc_e(kk, p0)
            return 0

        lax.fori_loop(0, n_t, loop_body, 0)

    return k(nodes2, ne2, src_tab, dst_tab, w_tab)


def _mlp_block(agg_ref, nodes_ref, w1_ref, w2_ref, w3_ref, w4_ref,
               b1_ref, g1_ref, e1_ref, b2_ref, g2_ref, e2_ref,
               b3_ref, g3_ref, e3_ref, b4_ref, g4_ref, e4_ref, out_ref):
    hi = jax.lax.Precision.DEFAULT
    D = nodes_ref.shape[-1]

    def ln_tanh(x, b_r, g_r, e_r):
        x = x + b_r[0]
        m = jnp.mean(x, axis=-1, keepdims=True)
        v = jnp.mean((x - m) * (x - m), axis=-1, keepdims=True)
        y = (x - m) * lax.rsqrt(v + 1e-5) * g_r[0] + e_r[0]
        return jnp.tanh(y)

    a_in = agg_ref[0, 0]
    a_out = agg_ref[0, 1]
    nd = nodes_ref[0]
    w1 = w1_ref[...]
    x = (jnp.dot(a_in, w1[0:D, :], precision=hi)
         + jnp.dot(a_out, w1[D:2 * D, :], precision=hi)
         + jnp.dot(nd, w1[2 * D:3 * D, :], precision=hi))
    x = ln_tanh(x, b1_ref, g1_ref, e1_ref)
    x = ln_tanh(jnp.dot(x, w2_ref[...], precision=hi), b2_ref, g2_ref, e2_ref)
    x = ln_tanh(jnp.dot(x, w3_ref[...], precision=hi), b3_ref, g3_ref, e3_ref)
    x = ln_tanh(jnp.dot(x, w4_ref[...], precision=hi), b4_ref, g4_ref, e4_ref)
    out_ref[0] = x


def _tc_mlp(agg4, nodes, w1t, w2t, w3t, w4t, vecs):
    B, N, D = nodes.shape
    R = 2000 if N % 2000 == 0 else 1000
    grid = (B, N // R)
    vspec = pl.BlockSpec((1, D), lambda b, i: (0, 0))
    return pl.pallas_call(
        _mlp_block,
        grid=grid,
        in_specs=[
            pl.BlockSpec((1, 2, R, D), lambda b, i: (b, 0, i, 0)),
            pl.BlockSpec((1, R, D), lambda b, i: (b, i, 0)),
            pl.BlockSpec((3 * D, D), lambda b, i: (0, 0)),
            pl.BlockSpec((D, D), lambda b, i: (0, 0)),
            pl.BlockSpec((D, D), lambda b, i: (0, 0)),
            pl.BlockSpec((D, D), lambda b, i: (0, 0)),
        ] + [vspec] * 12,
        out_specs=pl.BlockSpec((1, R, D), lambda b, i: (b, i, 0)),
        out_shape=jax.ShapeDtypeStruct((B, N, D), jnp.float32),
    )(agg4, nodes, w1t, w2t, w3t, w4t, *vecs)


def kernel(nodes, node_edges, edges, edge_weights,
           W1, b1, g1, be1, W2, b2, g2, be2,
           W3, b3, g3, be3, W4, b4, g4, be4):
    B, N, D = nodes.shape
    E = edges.shape[1]
    num_units = B * 2 * N

    boff = (jnp.arange(B, dtype=jnp.int32) * jnp.int32(N))[:, None, None]
    ids = edges + boff
    src_tab = ids[:, :, 0].reshape(B * E)
    dst_tab = ids[:, :, 1].reshape(B * E)
    w_tab = edge_weights.reshape(B * E)

    eoff = (jnp.arange(B, dtype=jnp.int32) * jnp.int32(E))[:, None, None, None]

    nodes2 = nodes.reshape(B * N, D)
    vecs = [v.reshape(1, D) for v in
            (b1, g1, be1, b2, g2, be2, b3, g3, be3, b4, g4, be4)]
    ne4 = (node_edges + eoff)

    nh = N // 2
    outs = []
    for h in range(2):
        lo = h * nh
        ne2h = ne4[:, :, lo:lo + nh, :].reshape(B * 2 * nh * _DEG // 128, 128)
        aggh = _sc_aggregate(nodes2, ne2h, src_tab, dst_tab, w_tab,
                             B * 2 * nh, nh)
        agg4h = aggh.reshape(B, 2, nh, D)
        outs.append(_tc_mlp(agg4h, nodes[:, lo:lo + nh], W1.T, W2.T, W3.T,
                            W4.T, vecs))
    return jnp.concatenate(outs, axis=1)


# speedup vs baseline: 1.0265x; 1.0265x over previous
import functools

import jax
import jax.numpy as jnp
from jax import lax
from jax.experimental import pallas as pl
from jax.experimental.pallas import tpu as pltpu
from jax.experimental.pallas import tpu_sc as plsc

_NC = 2
_NS = 16
_L = 16
_NW = _NC * _NS

_CHUNK = 8
_DEG = 32
_SLOTS = _CHUNK * _DEG
_G = _SLOTS // 128


def _sc_aggregate(nodes2, ne2, src_tab, dst_tab, w_tab, num_units, n_per_dir):
    D = nodes2.shape[1]
    nvec = D // _L
    num_chunks = num_units // _CHUNK
    chunks_per_dir = n_per_dir // _CHUNK
    max_k = (num_chunks + _NW - 1) // _NW + 2
    n_t = (max_k + 1) // 2 + 1

    mesh = plsc.VectorSubcoreMesh(
        core_axis_name="c", subcore_axis_name="s",
        num_cores=_NC, num_subcores=_NS)

    @functools.partial(
        pl.kernel,
        out_type=jax.ShapeDtypeStruct((num_units, D), jnp.float32),
        mesh=mesh,
        scratch_types=[
            pltpu.VMEM((2, _G, 128), jnp.int32),
            pltpu.VMEM((2, _G, 128), jnp.int32),
            pltpu.VMEM((2, _SLOTS), jnp.float32),
            pltpu.VMEM((2, _SLOTS), jnp.float32),
            pltpu.VMEM((2, _SLOTS, 128), jnp.float32),
            pltpu.VMEM((2, _CHUNK, 128), jnp.float32),
        ] + [pltpu.SemaphoreType.DMA] * 8,
    )
    def k(nodes_hbm, ne_hbm, src_hbm, dst_hbm, w_hbm, out_hbm,
          eidx_v, ids_v, wland_v, wacc_v, rows_v, out_v,
          sA0, sA1, sB0, sB1, sC0, sC1, sE0, sE1):
        wid = lax.axis_index("s") * _NC + lax.axis_index("c")
        sA = (sA0, sA1)
        sB = (sB0, sB1)
        sC = (sC0, sC1)
        sE = (sE0, sE1)

        def cid_of(kk):
            return wid + kk * _NW

        def local_valid(kk):
            return cid_of(kk) < num_chunks

        def stage_a(kk, p):
            cid = cid_of(kk)

            @pl.when(local_valid(kk))
            def _():
                pltpu.async_copy(
                    ne_hbm.at[pl.ds(cid * _G, _G), :], eidx_v.at[p], sA[p])

        def wait_a(kk, p):
            @pl.when(local_valid(kk))
            def _():
                pltpu.make_async_copy(
                    ne_hbm.at[pl.ds(0, _G), :], eidx_v.at[p], sA[p]).wait()

        def stage_b(kk, p):
            cid = cid_of(kk)
            valid = local_valid(kk)
            dcol = (cid // chunks_per_dir) % 2

            def issue(tab_hbm):
                for g in range(_G):
                    pltpu.async_copy(
                        tab_hbm.at[eidx_v.at[p, g]], ids_v.at[p, g], sB[p])
                for g in range(_G):
                    pltpu.async_copy(
                        w_hbm.at[eidx_v.at[p, g]],
                        wland_v.at[p, pl.ds(g * 128, 128)], sB[p])

            @pl.when(valid & (dcol == 0))
            def _():
                issue(src_hbm)

            @pl.when(valid & (dcol == 1))
            def _():
                issue(dst_hbm)

        def wait_b(kk, p):
            @pl.when(local_valid(kk))
            def _():
                pltpu.make_async_copy(
                    ne_hbm.at[pl.ds(0, _G), :], ids_v.at[p], sB[p]).wait()
                pltpu.make_async_copy(
                    w_hbm.at[pl.ds(0, _SLOTS)], wland_v.at[p],
                    sB[p]).wait()

        def wcopy(kk, p):
            @pl.when(local_valid(kk))
            def _():
                for j in range(_SLOTS // _L):
                    wacc_v[p, pl.ds(j * _L, _L)] = (
                        wland_v[p, pl.ds(j * _L, _L)])

        def stage_c(kk, p):
            @pl.when(local_valid(kk))
            def _():
                for g in range(_G):
                    pltpu.async_copy(
                        nodes_hbm.at[ids_v.at[p, g]],
                        rows_v.at[p, pl.ds(g * 128, 128), :], sC[p])

        def wait_c(kk, p):
            @pl.when(local_valid(kk))
            def _():
                pltpu.make_async_copy(
                    nodes_hbm.at[pl.ds(0, _SLOTS), :],
                    rows_v.at[p], sC[p]).wait()

        def wait_e(kk, p, extra_pred):
            @pl.when(extra_pred & local_valid(kk))
            def _():
                pltpu.make_async_copy(
                    out_hbm.at[pl.ds(0, _CHUNK), :], out_v.at[p],
                    sE[p]).wait()

        def acc_e(kk, p):
            cid = cid_of(kk)

            @pl.when(local_valid(kk))
            def _():
                def unit_body(i, _):
                    wa = wacc_v[p, pl.ds(i * _DEG, _L)]
                    wb = wacc_v[p, pl.ds(i * _DEG + _L, _L)]

                    def edge_body(d, acc):
                        s0 = i * _DEG + d
                        iv = jnp.broadcast_to(d, (_L,))
                        wva = jnp.take_along_axis(
                            wa, iv, axis=0, mode="promise_in_bounds")
                        wvb = jnp.take_along_axis(
                            wb, iv, axis=0, mode="promise_in_bounds")
                        return tuple(
                            acc[v]
                            + wva * rows_v[p, s0, pl.ds(v * _L, _L)]
                            + wvb * rows_v[p, s0 + _L, pl.ds(v * _L, _L)]
                            for v in range(nvec))
                    zero = jnp.zeros((_L,), jnp.float32)
                    acc = lax.fori_loop(
                        0, _L, edge_body, tuple(zero for _ in range(nvec)),
                        unroll=4)
                    for v in range(nvec):
                        out_v[p, i, pl.ds(v * _L, _L)] = acc[v]
                    return 0
                lax.fori_loop(0, _CHUNK, unit_body, 0)
                pltpu.async_copy(
                    out_v.at[p],
                    out_hbm.at[pl.ds(cid * _CHUNK, _CHUNK), :], sE[p])

        stage_a(0, 0)
        stage_a(1, 1)
        wait_a(0, 0)
        stage_b(0, 0)
        wait_b(0, 0)
        wcopy(0, 0)
        stage_c(0, 0)
        wait_a(1, 1)
        stage_b(1, 1)

        def loop_body(t, _):
            for m in range(2):
                kk = 2 * t + m
                p0 = m
                p1 = 1 - m
                stage_a(kk + 2, p0)
                wait_b(kk + 1, p1)
                wcopy(kk + 1, p1)
                stage_c(kk + 1, p1)
                wait_c(kk, p0)
                wait_a(kk + 2, p0)
                stage_b(kk + 2, p0)
                wait_e(kk - 2, p0, t >= 1)
                acc_e(kk, p0)
            return 0

        lax.fori_loop(0, n_t, loop_body, 0)

    return k(nodes2, ne2, src_tab, dst_tab, w_tab)


def _mlp_block(agg_ref, nodes_ref, w1_ref, w2_ref, w3_ref, w4_ref,
               b1_ref, g1_ref, e1_ref, b2_ref, g2_ref, e2_ref,
               b3_ref, g3_ref, e3_ref, b4_ref, g4_ref, e4_ref, out_ref):
    hi = jax.lax.Precision.DEFAULT
    D = nodes_ref.shape[-1]

    def ln_tanh(x, b_r, g_r, e_r):
        x = x + b_r[0]
        m = jnp.mean(x, axis=-1, keepdims=True)
        v = jnp.mean((x - m) * (x - m), axis=-1, keepdims=True)
        y = (x - m) * lax.rsqrt(v + 1e-5) * g_r[0] + e_r[0]
        return jnp.tanh(y)

    a_in = agg_ref[0, 0]
    a_out = agg_ref[0, 1]
    nd = nodes_ref[0]
    w1 = w1_ref[...]
    x = (jnp.dot(a_in, w1[0:D, :], precision=hi)
         + jnp.dot(a_out, w1[D:2 * D, :], precision=hi)
         + jnp.dot(nd, w1[2 * D:3 * D, :], precision=hi))
    x = ln_tanh(x, b1_ref, g1_ref, e1_ref)
    x = ln_tanh(jnp.dot(x, w2_ref[...], precision=hi), b2_ref, g2_ref, e2_ref)
    x = ln_tanh(jnp.dot(x, w3_ref[...], precision=hi), b3_ref, g3_ref, e3_ref)
    x = ln_tanh(jnp.dot(x, w4_ref[...], precision=hi), b4_ref, g4_ref, e4_ref)
    out_ref[0] = x


def _tc_mlp(agg4, nodes, w1t, w2t, w3t, w4t, vecs):
    B, N, D = nodes.shape
    R = 2000 if N % 2000 == 0 else 1000
    grid = (B, N // R)
    vspec = pl.BlockSpec((1, D), lambda b, i: (0, 0))
    return pl.pallas_call(
        _mlp_block,
        grid=grid,
        in_specs=[
            pl.BlockSpec((1, 2, R, D), lambda b, i: (b, 0, i, 0)),
            pl.BlockSpec((1, R, D), lambda b, i: (b, i, 0)),
            pl.BlockSpec((3 * D, D), lambda b, i: (0, 0)),
            pl.BlockSpec((D, D), lambda b, i: (0, 0)),
            pl.BlockSpec((D, D), lambda b, i: (0, 0)),
            pl.BlockSpec((D, D), lambda b, i: (0, 0)),
        ] + [vspec] * 12,
        out_specs=pl.BlockSpec((1, R, D), lambda b, i: (b, i, 0)),
        out_shape=jax.ShapeDtypeStruct((B, N, D), jnp.float32),
    )(agg4, nodes, w1t, w2t, w3t, w4t, *vecs)


def kernel(nodes, node_edges, edges, edge_weights,
           W1, b1, g1, be1, W2, b2, g2, be2,
           W3, b3, g3, be3, W4, b4, g4, be4):
    B, N, D = nodes.shape
    E = edges.shape[1]
    num_units = B * 2 * N

    boff = (jnp.arange(B, dtype=jnp.int32) * jnp.int32(N))[:, None, None]
    ids = edges + boff
    src_tab = ids[:, :, 0].reshape(B * E)
    dst_tab = ids[:, :, 1].reshape(B * E)
    w_tab = edge_weights.reshape(B * E)

    eoff = (jnp.arange(B, dtype=jnp.int32) * jnp.int32(E))[:, None, None, None]

    nodes2 = nodes.reshape(B * N, D)
    ne2 = (node_edges + eoff).reshape(num_units * _DEG // 128, 128)
    agg = _sc_aggregate(nodes2, ne2, src_tab, dst_tab, w_tab,
                        num_units, N)
    agg4 = agg.reshape(B, 2, N, D)

    vecs = [v.reshape(1, D) for v in
            (b1, g1, be1, b2, g2, be2, b3, g3, be3, b4, g4, be4)]
    return _tc_mlp(agg4, nodes, W1.T, W2.T, W3.T, W4.T, vecs)


# speedup vs baseline: 1.0505x; 1.0234x over previous
import functools

import jax
import jax.numpy as jnp
from jax import lax
from jax.experimental import pallas as pl
from jax.experimental.pallas import tpu as pltpu
from jax.experimental.pallas import tpu_sc as plsc

_NC = 2
_NS = 16
_L = 16
_NW = _NC * _NS

_CHUNK = 8
_DEG = 32
_SLOTS = _CHUNK * _DEG
_G = _SLOTS // 128


def _sc_aggregate(nodes2, ne2, src_tab, dst_tab, w_tab, num_units, n_per_dir):
    D = nodes2.shape[1]
    nvec = D // _L
    num_chunks = num_units // _CHUNK
    chunks_per_dir = n_per_dir // _CHUNK
    max_k = (num_chunks + _NW - 1) // _NW + 3
    n_t = (max_k + 2) // 3 + 1

    mesh = plsc.VectorSubcoreMesh(
        core_axis_name="c", subcore_axis_name="s",
        num_cores=_NC, num_subcores=_NS)

    @functools.partial(
        pl.kernel,
        out_type=jax.ShapeDtypeStruct((num_units, D), jnp.float32),
        mesh=mesh,
        scratch_types=[
            pltpu.VMEM((3, _G, 128), jnp.int32),
            pltpu.VMEM((3, _G, 128), jnp.int32),
            pltpu.VMEM((3, _G, 128), jnp.float32),
            pltpu.VMEM((3, _G, 128), jnp.float32),
            pltpu.VMEM((3, _SLOTS, 128), jnp.float32),
            pltpu.VMEM((3, _CHUNK, 128), jnp.float32),
        ] + [pltpu.SemaphoreType.DMA] * 12,
    )
    def k(nodes_hbm, ne_hbm, src_hbm, dst_hbm, w_hbm, out_hbm,
          eidx_v, ids_v, wland_v, wacc_v, rows_v, out_v,
          sA0, sA1, sA2, sB0, sB1, sB2, sC0, sC1, sC2, sE0, sE1, sE2):
        wid = lax.axis_index("s") * _NC + lax.axis_index("c")
        sA = (sA0, sA1, sA2)
        sB = (sB0, sB1, sB2)
        sC = (sC0, sC1, sC2)
        sE = (sE0, sE1, sE2)

        def cid_of(kk):
            return wid + kk * _NW

        def local_valid(kk):
            return cid_of(kk) < num_chunks

        def stage_a(kk, p):
            cid = cid_of(kk)

            @pl.when(local_valid(kk))
            def _():
                pltpu.async_copy(
                    ne_hbm.at[pl.ds(cid * _G, _G), :], eidx_v.at[p], sA[p])

        def wait_a(kk, p):
            @pl.when(local_valid(kk))
            def _():
                pltpu.make_async_copy(
                    ne_hbm.at[pl.ds(0, _G), :], eidx_v.at[p], sA[p]).wait()

        def stage_b(kk, p):
            cid = cid_of(kk)
            valid = local_valid(kk)
            dcol = (cid // chunks_per_dir) % 2

            def issue(tab_hbm):
                for g in range(_G):
                    pltpu.async_copy(
                        tab_hbm.at[eidx_v.at[p, g]], ids_v.at[p, g], sB[p])
                for g in range(_G):
                    pltpu.async_copy(
                        w_hbm.at[eidx_v.at[p, g]],
                        wland_v.at[p, g], sB[p])

            @pl.when(valid & (dcol == 0))
            def _():
                issue(src_hbm)

            @pl.when(valid & (dcol == 1))
            def _():
                issue(dst_hbm)

        def wait_b(kk, p):
            @pl.when(local_valid(kk))
            def _():
                pltpu.make_async_copy(
                    ne_hbm.at[pl.ds(0, _G), :], ids_v.at[p], sB[p]).wait()
                pltpu.make_async_copy(
                    nodes_hbm.at[pl.ds(0, _G), pl.ds(0, 128)],
                    wland_v.at[p], sB[p]).wait()

        def wcopy(kk, p):
            @pl.when(local_valid(kk))
            def _():
                for j in range(_SLOTS // _L):
                    wacc_v[p, j // 8, pl.ds((j % 8) * _L, _L)] = (
                        wland_v[p, j // 8, pl.ds((j % 8) * _L, _L)])

        def stage_c(kk, p):
            @pl.when(local_valid(kk))
            def _():
                for g in range(_G):
                    pltpu.async_copy(
                        nodes_hbm.at[ids_v.at[p, g]],
                        rows_v.at[p, pl.ds(g * 128, 128), :], sC[p])

        def wait_c(kk, p):
            @pl.when(local_valid(kk))
            def _():
                pltpu.make_async_copy(
                    nodes_hbm.at[pl.ds(0, _SLOTS), :],
                    rows_v.at[p], sC[p]).wait()

        def wait_e(kk, p, extra_pred):
            @pl.when(extra_pred & local_valid(kk))
            def _():
                pltpu.make_async_copy(
                    out_hbm.at[pl.ds(0, _CHUNK), :], out_v.at[p],
                    sE[p]).wait()

        def acc_e(kk, p):
            cid = cid_of(kk)

            @pl.when(local_valid(kk))
            def _():
                def unit_body(i, _):
                    wa = wacc_v[p, i // 4, pl.ds((i % 4) * _DEG, _L)]
                    wb = wacc_v[p, i // 4, pl.ds((i % 4) * _DEG + _L, _L)]

                    def edge_body(d, acc):
                        s0 = i * _DEG + d
                        iv = jnp.broadcast_to(d, (_L,))
                        wva = jnp.take_along_axis(
                            wa, iv, axis=0, mode="promise_in_bounds")
                        wvb = jnp.take_along_axis(
                            wb, iv, axis=0, mode="promise_in_bounds")
                        return tuple(
                            acc[v]
                            + wva * rows_v[p, s0, pl.ds(v * _L, _L)]
                            + wvb * rows_v[p, s0 + _L, pl.ds(v * _L, _L)]
                            for v in range(nvec))
                    zero = jnp.zeros((_L,), jnp.float32)
                    acc = lax.fori_loop(
                        0, _L, edge_body, tuple(zero for _ in range(nvec)),
                        unroll=4)
                    for v in range(nvec):
                        out_v[p, i, pl.ds(v * _L, _L)] = acc[v]
                    return 0
                lax.fori_loop(0, _CHUNK, unit_body, 0)
                pltpu.async_copy(
                    out_v.at[p],
                    out_hbm.at[pl.ds(cid * _CHUNK, _CHUNK), :], sE[p])

        for j in range(3):
            stage_a(j, j)
            wait_a(j, j)
            stage_b(j, j)
        for j in range(2):
            wait_b(j, j)
            wcopy(j, j)
            stage_c(j, j)

        def loop_body(t, _):
            for m in range(3):
                kk = 3 * t + m
                p0 = m
                p2 = (m + 2) % 3
                stage_a(kk + 3, p0)
                wait_b(kk + 2, p2)
                wcopy(kk + 2, p2)
                stage_c(kk + 2, p2)
                wait_c(kk, p0)
                wait_a(kk + 3, p0)
                stage_b(kk + 3, p0)
                wait_e(kk - 3, p0, t >= 1)
                acc_e(kk, p0)
            return 0

        lax.fori_loop(0, n_t, loop_body, 0)

    return k(nodes2, ne2, src_tab, dst_tab, w_tab)


def _mlp_block(agg_ref, nodes_ref, w1_ref, w2_ref, w3_ref, w4_ref,
               b1_ref, g1_ref, e1_ref, b2_ref, g2_ref, e2_ref,
               b3_ref, g3_ref, e3_ref, b4_ref, g4_ref, e4_ref, out_ref):
    hi = jax.lax.Precision.DEFAULT
    D = nodes_ref.shape[-1]

    def ln_tanh(x, b_r, g_r, e_r):
        x = x + b_r[0]
        m = jnp.mean(x, axis=-1, keepdims=True)
        v = jnp.mean((x - m) * (x - m), axis=-1, keepdims=True)
        y = (x - m) * lax.rsqrt(v + 1e-5) * g_r[0] + e_r[0]
        return jnp.tanh(y)

    a_in = agg_ref[0, 0]
    a_out = agg_ref[0, 1]
    nd = nodes_ref[0]
    w1 = w1_ref[...]
    x = (jnp.dot(a_in, w1[0:D, :], precision=hi)
         + jnp.dot(a_out, w1[D:2 * D, :], precision=hi)
         + jnp.dot(nd, w1[2 * D:3 * D, :], precision=hi))
    x = ln_tanh(x, b1_ref, g1_ref, e1_ref)
    x = ln_tanh(jnp.dot(x, w2_ref[...], precision=hi), b2_ref, g2_ref, e2_ref)
    x = ln_tanh(jnp.dot(x, w3_ref[...], precision=hi), b3_ref, g3_ref, e3_ref)
    x = ln_tanh(jnp.dot(x, w4_ref[...], precision=hi), b4_ref, g4_ref, e4_ref)
    out_ref[0] = x


def _tc_mlp(agg4, nodes, w1t, w2t, w3t, w4t, vecs):
    B, N, D = nodes.shape
    R = 2000 if N % 2000 == 0 else 1000
    grid = (B, N // R)
    vspec = pl.BlockSpec((1, D), lambda b, i: (0, 0))
    return pl.pallas_call(
        _mlp_block,
        grid=grid,
        in_specs=[
            pl.BlockSpec((1, 2, R, D), lambda b, i: (b, 0, i, 0)),
            pl.BlockSpec((1, R, D), lambda b, i: (b, i, 0)),
            pl.BlockSpec((3 * D, D), lambda b, i: (0, 0)),
            pl.BlockSpec((D, D), lambda b, i: (0, 0)),
            pl.BlockSpec((D, D), lambda b, i: (0, 0)),
            pl.BlockSpec((D, D), lambda b, i: (0, 0)),
        ] + [vspec] * 12,
        out_specs=pl.BlockSpec((1, R, D), lambda b, i: (b, i, 0)),
        out_shape=jax.ShapeDtypeStruct((B, N, D), jnp.float32),
    )(agg4, nodes, w1t, w2t, w3t, w4t, *vecs)


def kernel(nodes, node_edges, edges, edge_weights,
           W1, b1, g1, be1, W2, b2, g2, be2,
           W3, b3, g3, be3, W4, b4, g4, be4):
    B, N, D = nodes.shape
    E = edges.shape[1]
    num_units = B * 2 * N

    boff = (jnp.arange(B, dtype=jnp.int32) * jnp.int32(N))[:, None, None]
    ids = edges + boff
    src_tab = ids[:, :, 0].reshape(B * E)
    dst_tab = ids[:, :, 1].reshape(B * E)
    w_tab = edge_weights.reshape(B * E)

    eoff = (jnp.arange(B, dtype=jnp.int32) * jnp.int32(E))[:, None, None, None]

    nodes2 = nodes.reshape(B * N, D)
    ne2 = (node_edges + eoff).reshape(num_units * _DEG // 128, 128)
    agg = _sc_aggregate(nodes2, ne2, src_tab, dst_tab, w_tab,
                        num_units, N)
    agg4 = agg.reshape(B, 2, N, D)

    vecs = [v.reshape(1, D) for v in
            (b1, g1, be1, b2, g2, be2, b3, g3, be3, b4, g4, be4)]
    return _tc_mlp(agg4, nodes, W1.T, W2.T, W3.T, W4.T, vecs)
